# Initial kernel scaffold; baseline (speedup 1.0000x reference)
import jax
import jax.numpy as jnp
from jax.experimental import pallas as pl


def kernel(points, W, gamma, beta):
    raise NotImplementedError("write your pallas kernel here")


# speedup vs baseline: 1.2594x; 1.2594x over previous
import jax
import jax.numpy as jnp
from jax.experimental import pallas as pl
from jax.experimental.pallas import tpu as pltpu

_B = 2
_GX = 512
_GY = 512
_NV = _GX * _GY
_D = 64
_VOXEL = 0.2
_PCMIN = -51.2

_PBLK = 4096
_GBLK = 8


def _matmul_body(pts_ref, mg_ref, w_ref, x_ref):
    p = pts_ref[...]
    m = mg_ref[...]
    xy = p[:, 1:3]
    cf = (xy - _PCMIN) / _VOXEL
    coords = jnp.floor(cf)
    center = coords * _VOXEL + _VOXEL / 2.0 + _PCMIN
    feats = jnp.concatenate([p[:, 1:5], p[:, 1:4] - m, xy - center], axis=-1)
    x_ref[...] = jnp.dot(feats, w_ref[...].T, preferred_element_type=jnp.float32)


def _finalize_body(cin_ref, cnt_ref, s_ref, b_ref, out_ref):
    v = cin_ref[0]
    mask = cnt_ref[0] > 0.0
    s = s_ref[0]
    bb = b_ref[0]
    r = jnp.maximum(v * s + bb, 0.0)
    r = jnp.where(mask, r, 0.0)
    vr = r.reshape(_GBLK * _GX, _D)
    t = vr.T
    out_ref[0] = t.reshape(_D, _GBLK, _GX)


def kernel(points, W, gamma, beta):
    n = points.shape[0]
    xy = points[:, 1:3]
    coords = ((xy - _PCMIN) / _VOXEL).astype(jnp.int32)
    bidx = points[:, 0].astype(jnp.int32)
    pidx = bidx * _NV + coords[:, 1] * _GX + coords[:, 0]

    ones = jnp.ones((n,), dtype=jnp.float32)
    cnt = jax.ops.segment_sum(ones, pidx, num_segments=_B * _NV)
    sums = jax.ops.segment_sum(points[:, 1:4], pidx, num_segments=_B * _NV)
    mean = sums / jnp.maximum(cnt, 1.0)[:, None]
    mean_g = mean[pidx]

    x = pl.pallas_call(
        _matmul_body,
        grid=(n // _PBLK,),
        in_specs=[
            pl.BlockSpec((_PBLK, 5), lambda i: (i, 0)),
            pl.BlockSpec((_PBLK, 3), lambda i: (i, 0)),
            pl.BlockSpec((_D, 9), lambda i: (0, 0)),
        ],
        out_specs=pl.BlockSpec((_PBLK, _D), lambda i: (i, 0)),
        out_shape=jax.ShapeDtypeStruct((n, _D), jnp.float32),
    )(points, mean_g, W)

    mu = jnp.mean(x, axis=0)
    var = jnp.mean((x - mu) ** 2, axis=0)
    s = gamma / jnp.sqrt(var + 1e-3)
    bb = beta - mu * s

    seg_max = jax.ops.segment_max(x, pidx, num_segments=_B * _NV)
    cgrid = seg_max.reshape(_B, _GY, _GX, _D)
    cntg = cnt.reshape(_B, _GY, _GX, 1)

    canvas = pl.pallas_call(
        _finalize_body,
        grid=(_B, _GY // _GBLK),
        in_specs=[
            pl.BlockSpec((1, _GBLK, _GX, _D), lambda b, g: (b, g, 0, 0)),
            pl.BlockSpec((1, _GBLK, _GX, 1), lambda b, g: (b, g, 0, 0)),
            pl.BlockSpec((1, _D), lambda b, g: (0, 0)),
            pl.BlockSpec((1, _D), lambda b, g: (0, 0)),
        ],
        out_specs=pl.BlockSpec((1, _D, _GBLK, _GX), lambda b, g: (b, 0, g, 0)),
        out_shape=jax.ShapeDtypeStruct((_B, _D, _GY, _GX), jnp.float32),
    )(cgrid, cntg, s.reshape(1, _D), bb.reshape(1, _D))

    return canvas


# speedup vs baseline: 1.4429x; 1.1457x over previous
import jax
import jax.numpy as jnp
from jax.experimental import pallas as pl
from jax.experimental.pallas import tpu as pltpu

_B = 2
_GX = 512
_GY = 512
_NV = _GX * _GY
_D = 64
_VOXEL = 0.2
_PCMIN = -51.2

_PBLK = 4096
_GBLK = 8


def _matmul_body(pts_ref, mg_ref, w_ref, x_ref):
    p = pts_ref[...]
    sg = mg_ref[...]
    m = sg[:, 0:3] / jnp.maximum(sg[:, 3:4], 1.0)
    xy = p[:, 1:3]
    cf = (xy - _PCMIN) / _VOXEL
    coords = jnp.floor(cf)
    center = coords * _VOXEL + _VOXEL / 2.0 + _PCMIN
    feats = jnp.concatenate([p[:, 1:5], p[:, 1:4] - m, xy - center], axis=-1)
    x_ref[...] = jnp.dot(feats, w_ref[...].T, preferred_element_type=jnp.float32)


def _finalize_body(cin_ref, cnt_ref, s_ref, b_ref, out_ref):
    v = cin_ref[0]
    mask = cnt_ref[0] > 0.0
    s = s_ref[0]
    bb = b_ref[0]
    r = jnp.maximum(v * s + bb, 0.0)
    r = jnp.where(mask, r, 0.0)
    vr = r.reshape(_GBLK * _GX, _D)
    t = vr.T
    out_ref[0] = t.reshape(_D, _GBLK, _GX)


def kernel(points, W, gamma, beta):
    n = points.shape[0]
    xy = points[:, 1:3]
    coords = ((xy - _PCMIN) / _VOXEL).astype(jnp.int32)
    bidx = points[:, 0].astype(jnp.int32)
    pidx = bidx * _NV + coords[:, 1] * _GX + coords[:, 0]

    vals = jnp.concatenate(
        [points[:, 1:4], jnp.ones((n, 1), dtype=jnp.float32)], axis=1)
    table = jax.ops.segment_sum(vals, pidx, num_segments=_B * _NV)
    mean_g = table[pidx]
    cnt = table[:, 3]

    x = pl.pallas_call(
        _matmul_body,
        grid=(n // _PBLK,),
        in_specs=[
            pl.BlockSpec((_PBLK, 5), lambda i: (i, 0)),
            pl.BlockSpec((_PBLK, 4), lambda i: (i, 0)),
            pl.BlockSpec((_D, 9), lambda i: (0, 0)),
        ],
        out_specs=pl.BlockSpec((_PBLK, _D), lambda i: (i, 0)),
        out_shape=jax.ShapeDtypeStruct((n, _D), jnp.float32),
    )(points, mean_g, W)

    mu = jnp.mean(x, axis=0)
    var = jnp.mean((x - mu) ** 2, axis=0)
    s = gamma / jnp.sqrt(var + 1e-3)
    bb = beta - mu * s

    seg_max = jax.ops.segment_max(x, pidx, num_segments=_B * _NV)
    cgrid = seg_max.reshape(_B, _GY, _GX, _D)
    cntg = cnt.reshape(_B, _GY, _GX, 1)

    canvas = pl.pallas_call(
        _finalize_body,
        grid=(_B, _GY // _GBLK),
        in_specs=[
            pl.BlockSpec((1, _GBLK, _GX, _D), lambda b, g: (b, g, 0, 0)),
            pl.BlockSpec((1, _GBLK, _GX, 1), lambda b, g: (b, g, 0, 0)),
            pl.BlockSpec((1, _D), lambda b, g: (0, 0)),
            pl.BlockSpec((1, _D), lambda b, g: (0, 0)),
        ],
        out_specs=pl.BlockSpec((1, _D, _GBLK, _GX), lambda b, g: (b, 0, g, 0)),
        out_shape=jax.ShapeDtypeStruct((_B, _D, _GY, _GX), jnp.float32),
    )(cgrid, cntg, s.reshape(1, _D), bb.reshape(1, _D))

    return canvas


# speedup vs baseline: 1.5019x; 1.0409x over previous
import jax
import jax.numpy as jnp
from jax.experimental import pallas as pl
from jax.experimental.pallas import tpu as pltpu

_B = 2
_GX = 512
_GY = 512
_NV = _GX * _GY
_D = 64
_VOXEL = 0.2
_PCMIN = -51.2

_PBLK = 4096
_GBLK = 8


def _matmul_body(pts_ref, mg_ref, w_ref, x_ref):
    p = pts_ref[...]
    m = mg_ref[...]
    xy = p[:, 1:3]
    cf = (xy - _PCMIN) / _VOXEL
    coords = jnp.floor(cf)
    center = coords * _VOXEL + _VOXEL / 2.0 + _PCMIN
    feats = jnp.concatenate([p[:, 1:5], p[:, 1:4] - m, xy - center], axis=-1)
    x_ref[...] = jnp.dot(feats, w_ref[...].T, preferred_element_type=jnp.float32)


def _finalize_body(cin_ref, cnt_ref, s_ref, b_ref, out_ref):
    v = cin_ref[0]
    mask = cnt_ref[0] > 0.0
    s = s_ref[0]
    bb = b_ref[0]
    r = jnp.maximum(v * s + bb, 0.0)
    r = jnp.where(mask, r, 0.0)
    vr = r.reshape(_GBLK * _GX, _D)
    t = vr.T
    out_ref[0] = t.reshape(_D, _GBLK, _GX)


def kernel(points, W, gamma, beta):
    n = points.shape[0]
    xy = points[:, 1:3]
    coords = ((xy - _PCMIN) / _VOXEL).astype(jnp.int32)
    bidx = points[:, 0].astype(jnp.int32)
    pidx = bidx * _NV + coords[:, 1] * _GX + coords[:, 0]

    vals = jnp.concatenate(
        [points[:, 1:4], jnp.ones((n, 1), dtype=jnp.float32),
         jnp.zeros((n, 4), dtype=jnp.float32)], axis=1)
    table = jax.ops.segment_sum(vals, pidx, num_segments=_B * _NV)
    cnt = table[:, 3]
    mean = table[:, 0:3] / jnp.maximum(cnt, 1.0)[:, None]
    mean_g = mean[pidx]

    x = pl.pallas_call(
        _matmul_body,
        grid=(n // _PBLK,),
        in_specs=[
            pl.BlockSpec((_PBLK, 5), lambda i: (i, 0)),
            pl.BlockSpec((_PBLK, 3), lambda i: (i, 0)),
            pl.BlockSpec((_D, 9), lambda i: (0, 0)),
        ],
        out_specs=pl.BlockSpec((_PBLK, _D), lambda i: (i, 0)),
        out_shape=jax.ShapeDtypeStruct((n, _D), jnp.float32),
    )(points, mean_g, W)

    mu = jnp.mean(x, axis=0)
    var = jnp.mean((x - mu) ** 2, axis=0)
    s = gamma / jnp.sqrt(var + 1e-3)
    bb = beta - mu * s

    seg_max = jax.ops.segment_max(x, pidx, num_segments=_B * _NV)
    cgrid = seg_max.reshape(_B, _GY, _GX, _D)
    cntg = cnt.reshape(_B, _GY, _GX, 1)

    canvas = pl.pallas_call(
        _finalize_body,
        grid=(_B, _GY // _GBLK),
        in_specs=[
            pl.BlockSpec((1, _GBLK, _GX, _D), lambda b, g: (b, g, 0, 0)),
            pl.BlockSpec((1, _GBLK, _GX, 1), lambda b, g: (b, g, 0, 0)),
            pl.BlockSpec((1, _D), lambda b, g: (0, 0)),
            pl.BlockSpec((1, _D), lambda b, g: (0, 0)),
        ],
        out_specs=pl.BlockSpec((1, _D, _GBLK, _GX), lambda b, g: (b, 0, g, 0)),
        out_shape=jax.ShapeDtypeStruct((_B, _D, _GY, _GX), jnp.float32),
    )(cgrid, cntg, s.reshape(1, _D), bb.reshape(1, _D))

    return canvas


# speedup vs baseline: 1.6608x; 1.1058x over previous
import functools

import jax
import jax.numpy as jnp
from jax import lax
from jax.experimental import pallas as pl
from jax.experimental.pallas import tpu as pltpu
from jax.experimental.pallas import tpu_sc as plsc

_B = 2
_GX = 512
_GY = 512
_NV = _GX * _GY
_D = 64
_VOXEL = 0.2
_PCMIN = -51.2

_PBLK = 4096
_GBLK = 8

_NPAD = 401408
_HALF = _B * _NV // 2
_TRASH = _HALF
_HROWS = _HALF + 256
_CH = 1568
_SUB = 112
_NCHUNK = _NPAD // 16 // _CH
_ZROWS = _HROWS // 16 // 4


def _seg_sum_body(pb_hbm, px_hbm, py_hbm, pz_hbm, tbl_hbm, pidx_hbm,
                  bbuf, vx, vy, vz, vo, zbuf, idxbuf, pidxbuf,
                  tx, ty, tz, tc):
    c = lax.axis_index("c")
    s = lax.axis_index("s")
    iota = lax.iota(jnp.int32, 16)
    ones16 = jnp.full((16,), 1.0, dtype=jnp.float32)
    zeros16 = jnp.zeros((16,), dtype=jnp.float32)

    def ones_body(g, _):
        vo[pl.ds(g * 16, 16)] = ones16
        return 0

    lax.fori_loop(0, _CH // 16, ones_body, 0)

    def zeros_body(g, _):
        zbuf[pl.ds(g * 16, 16)] = zeros16
        return 0

    lax.fori_loop(0, _HROWS // 16 // 16, zeros_body, 0)

    zoff = s * (_HROWS // 16)
    for t in (tx, ty, tz, tc):
        pltpu.sync_copy(zbuf, t.at[pl.ds(zoff, _HROWS // 16)])
    plsc.subcore_barrier()

    base = s * (_NPAD // 16)
    qbase = c * _HALF

    def chunk_body(ci, _):
        start = base + ci * _CH
        sl_in = pl.ds(start, _CH)
        pltpu.sync_copy(pb_hbm.at[sl_in], bbuf)
        pltpu.sync_copy(px_hbm.at[sl_in], vx)
        pltpu.sync_copy(py_hbm.at[sl_in], vy)
        pltpu.sync_copy(pz_hbm.at[sl_in], vz)

        def group_body(g, _):
            sl16 = pl.ds(g * 16, 16)
            b = bbuf[sl16]
            x = vx[sl16]
            y = vy[sl16]
            cx = ((x - _PCMIN) / _VOXEL).astype(jnp.int32)
            cy = ((y - _PCMIN) / _VOXEL).astype(jnp.int32)
            bi = b.astype(jnp.int32)
            pidx = bi * _NV + cy * _GX + cx
            local = pidx - qbase
            inr = (local >= 0) & (local < _HALF)
            localc = jnp.where(inr, local, _TRASH)
            pidxbuf[sl16] = pidx
            idxbuf[g // 7, pl.ds((g % 7) * 16, 16)] = localc
            return 0

        lax.fori_loop(0, _CH // 16, group_body, 0)

        for j in range(_CH // _SUB):
            sl = pl.ds(j * _SUB, _SUB)
            idxrow = idxbuf.at[j]
            pltpu.sync_copy(vx.at[sl], tx.at[idxrow], add=True)
            pltpu.sync_copy(vy.at[sl], ty.at[idxrow], add=True)
            pltpu.sync_copy(vz.at[sl], tz.at[idxrow], add=True)
            pltpu.sync_copy(vo.at[sl], tc.at[idxrow], add=True)

        @pl.when(c == 0)
        def _():
            pltpu.sync_copy(pidxbuf, pidx_hbm.at[pl.ds(start, _CH)])
        return 0

    lax.fori_loop(0, _NCHUNK, chunk_body, 0)
    plsc.subcore_barrier()

    wrows = _HALF // 16
    for k, t in enumerate((tx, ty, tz, tc)):
        pltpu.sync_copy(t.at[pl.ds(s * wrows, wrows)],
                        tbl_hbm.at[k, pl.ds(qbase + s * wrows, wrows)])


def _seg_sum(points_padded):
    mesh = plsc.VectorSubcoreMesh(core_axis_name="c", subcore_axis_name="s")
    f = functools.partial(
        pl.kernel,
        mesh=mesh,
        out_type=[
            jax.ShapeDtypeStruct((4, _B * _NV), jnp.float32),
            jax.ShapeDtypeStruct((_NPAD,), jnp.int32),
        ],
        scratch_types=[
            pltpu.VMEM((_CH,), jnp.float32),
            pltpu.VMEM((_CH,), jnp.float32),
            pltpu.VMEM((_CH,), jnp.float32),
            pltpu.VMEM((_CH,), jnp.float32),
            pltpu.VMEM((_CH,), jnp.float32),
            pltpu.VMEM((_HROWS // 16,), jnp.float32),
            pltpu.VMEM((_CH // _SUB, _SUB), jnp.int32),
            pltpu.VMEM((_CH,), jnp.int32),
            pltpu.VMEM_SHARED((_HROWS,), jnp.float32),
            pltpu.VMEM_SHARED((_HROWS,), jnp.float32),
            pltpu.VMEM_SHARED((_HROWS,), jnp.float32),
            pltpu.VMEM_SHARED((_HROWS,), jnp.float32),
        ],
    )(_seg_sum_body)
    return f(points_padded[:, 0], points_padded[:, 1],
             points_padded[:, 2], points_padded[:, 3])


def _matmul_body(pts_ref, mg_ref, w_ref, x_ref):
    p = pts_ref[...]
    m = mg_ref[...]
    xy = p[:, 1:3]
    cf = (xy - _PCMIN) / _VOXEL
    coords = jnp.floor(cf)
    center = coords * _VOXEL + _VOXEL / 2.0 + _PCMIN
    feats = jnp.concatenate([p[:, 1:5], p[:, 1:4] - m, xy - center], axis=-1)
    x_ref[...] = jnp.dot(feats, w_ref[...].T, preferred_element_type=jnp.float32)


def _finalize_body(cin_ref, cnt_ref, s_ref, b_ref, out_ref):
    v = cin_ref[0]
    mask = cnt_ref[0] > 0.0
    s = s_ref[0]
    bb = b_ref[0]
    r = jnp.maximum(v * s + bb, 0.0)
    r = jnp.where(mask, r, 0.0)
    vr = r.reshape(_GBLK * _GX, _D)
    t = vr.T
    out_ref[0] = t.reshape(_D, _GBLK, _GX)


def kernel(points, W, gamma, beta):
    n = points.shape[0]
    xy = points[:, 1:3]
    coords = ((xy - _PCMIN) / _VOXEL).astype(jnp.int32)
    bidx = points[:, 0].astype(jnp.int32)
    pidx = bidx * _NV + coords[:, 1] * _GX + coords[:, 0]

    pad = jnp.zeros((_NPAD - n, 5), dtype=jnp.float32).at[:, 0].set(4.0)
    points_padded = jnp.concatenate([points, pad], axis=0)
    table, _pidx_dump = _seg_sum(points_padded)
    cnt = table[3]
    mean = (table[0:3] / jnp.maximum(cnt, 1.0)[None, :]).T
    mean_g = mean[pidx]

    mean_g_pad = jnp.concatenate(
        [mean_g, jnp.zeros((_NPAD - n, 3), dtype=jnp.float32)], axis=0)
    x = pl.pallas_call(
        _matmul_body,
        grid=(_NPAD // _PBLK,),
        in_specs=[
            pl.BlockSpec((_PBLK, 5), lambda i: (i, 0)),
            pl.BlockSpec((_PBLK, 3), lambda i: (i, 0)),
            pl.BlockSpec((_D, 9), lambda i: (0, 0)),
        ],
        out_specs=pl.BlockSpec((_PBLK, _D), lambda i: (i, 0)),
        out_shape=jax.ShapeDtypeStruct((_NPAD, _D), jnp.float32),
    )(points_padded, mean_g_pad, W)
    x = x[:n]

    mu = jnp.mean(x, axis=0)
    var = jnp.mean((x - mu) ** 2, axis=0)
    s = gamma / jnp.sqrt(var + 1e-3)
    bb = beta - mu * s

    seg_max = jax.ops.segment_max(x, pidx, num_segments=_B * _NV)
    cgrid = seg_max.reshape(_B, _GY, _GX, _D)
    cntg = cnt.reshape(_B, _GY, _GX, 1)

    canvas = pl.pallas_call(
        _finalize_body,
        grid=(_B, _GY // _GBLK),
        in_specs=[
            pl.BlockSpec((1, _GBLK, _GX, _D), lambda b, g: (b, g, 0, 0)),
            pl.BlockSpec((1, _GBLK, _GX, 1), lambda b, g: (b, g, 0, 0)),
            pl.BlockSpec((1, _D), lambda b, g: (0, 0)),
            pl.BlockSpec((1, _D), lambda b, g: (0, 0)),
        ],
        out_specs=pl.BlockSpec((1, _D, _GBLK, _GX), lambda b, g: (b, 0, g, 0)),
        out_shape=jax.ShapeDtypeStruct((_B, _D, _GY, _GX), jnp.float32),
    )(cgrid, cntg, s.reshape(1, _D), bb.reshape(1, _D))

    return canvas


# speedup vs baseline: 1.6851x; 1.0146x over previous
import functools

import jax
import jax.numpy as jnp
from jax import lax
from jax.experimental import pallas as pl
from jax.experimental.pallas import tpu as pltpu
from jax.experimental.pallas import tpu_sc as plsc

_B = 2
_GX = 512
_GY = 512
_NV = _GX * _GY
_D = 64
_VOXEL = 0.2
_PCMIN = -51.2

_PBLK = 4096
_GBLK = 8

_NREAL = 400000
_NPAD = 401408
_HALF = _B * _NV // 2
_TRASH = _HALF
_HROWS = _HALF + 256
_CH = 1568
_SUB = 112
_NCHUNK = _NPAD // 16 // _CH
_ZROWS = _HROWS // 16 // 4


def _seg_sum_body(pb_hbm, px_hbm, py_hbm, pz_hbm, tbl_hbm, pidx_hbm,
                  bbuf, vx, vy, vz, vo, zbuf, idxbuf, pidxbuf,
                  tx, ty, tz, tc):
    c = lax.axis_index("c")
    s = lax.axis_index("s")
    iota = lax.iota(jnp.int32, 16)
    ones16 = jnp.full((16,), 1.0, dtype=jnp.float32)
    zeros16 = jnp.zeros((16,), dtype=jnp.float32)

    def ones_body(g, _):
        vo[pl.ds(g * 16, 16)] = ones16
        return 0

    lax.fori_loop(0, _CH // 16, ones_body, 0)

    def zeros_body(g, _):
        zbuf[pl.ds(g * 16, 16)] = zeros16
        return 0

    lax.fori_loop(0, _HROWS // 16 // 16, zeros_body, 0)

    zoff = s * (_HROWS // 16)
    for t in (tx, ty, tz, tc):
        pltpu.sync_copy(zbuf, t.at[pl.ds(zoff, _HROWS // 16)])
    plsc.subcore_barrier()

    base = s * (_NPAD // 16)
    qbase = c * _HALF

    def chunk_body(ci, _):
        start = base + ci * _CH
        sl_in = pl.ds(start, _CH)
        pltpu.sync_copy(pb_hbm.at[sl_in], bbuf)
        pltpu.sync_copy(px_hbm.at[sl_in], vx)
        pltpu.sync_copy(py_hbm.at[sl_in], vy)
        pltpu.sync_copy(pz_hbm.at[sl_in], vz)

        def group_body(g, _):
            sl16 = pl.ds(g * 16, 16)
            b = bbuf[sl16]
            x = vx[sl16]
            y = vy[sl16]
            cx = ((x - _PCMIN) / _VOXEL).astype(jnp.int32)
            cy = ((y - _PCMIN) / _VOXEL).astype(jnp.int32)
            bi = b.astype(jnp.int32)
            pidx = bi * _NV + cy * _GX + cx
            local = pidx - qbase
            inr = (local >= 0) & (local < _HALF)
            localc = jnp.where(inr, local, _TRASH)
            pidxbuf[sl16] = pidx
            idxbuf[g // 7, pl.ds((g % 7) * 16, 16)] = localc
            return 0

        lax.fori_loop(0, _CH // 16, group_body, 0)

        for j in range(_CH // _SUB):
            sl = pl.ds(j * _SUB, _SUB)
            idxrow = idxbuf.at[j]
            pltpu.sync_copy(vx.at[sl], tx.at[idxrow], add=True)
            pltpu.sync_copy(vy.at[sl], ty.at[idxrow], add=True)
            pltpu.sync_copy(vz.at[sl], tz.at[idxrow], add=True)
            pltpu.sync_copy(vo.at[sl], tc.at[idxrow], add=True)

        @pl.when(c == 0)
        def _():
            pltpu.sync_copy(pidxbuf, pidx_hbm.at[pl.ds(start, _CH)])
        return 0

    lax.fori_loop(0, _NCHUNK, chunk_body, 0)
    plsc.subcore_barrier()

    wrows = _HALF // 16
    for k, t in enumerate((tx, ty, tz, tc)):
        pltpu.sync_copy(t.at[pl.ds(s * wrows, wrows)],
                        tbl_hbm.at[k, pl.ds(qbase + s * wrows, wrows)])


def _seg_sum(points_padded):
    mesh = plsc.VectorSubcoreMesh(core_axis_name="c", subcore_axis_name="s")
    f = functools.partial(
        pl.kernel,
        mesh=mesh,
        out_type=[
            jax.ShapeDtypeStruct((4, _B * _NV), jnp.float32),
            jax.ShapeDtypeStruct((_NPAD,), jnp.int32),
        ],
        scratch_types=[
            pltpu.VMEM((_CH,), jnp.float32),
            pltpu.VMEM((_CH,), jnp.float32),
            pltpu.VMEM((_CH,), jnp.float32),
            pltpu.VMEM((_CH,), jnp.float32),
            pltpu.VMEM((_CH,), jnp.float32),
            pltpu.VMEM((_HROWS // 16,), jnp.float32),
            pltpu.VMEM((_CH // _SUB, _SUB), jnp.int32),
            pltpu.VMEM((_CH,), jnp.int32),
            pltpu.VMEM_SHARED((_HROWS,), jnp.float32),
            pltpu.VMEM_SHARED((_HROWS,), jnp.float32),
            pltpu.VMEM_SHARED((_HROWS,), jnp.float32),
            pltpu.VMEM_SHARED((_HROWS,), jnp.float32),
        ],
    )(_seg_sum_body)
    return f(points_padded[:, 0], points_padded[:, 1],
             points_padded[:, 2], points_padded[:, 3])


def _matmul_body(pts_ref, mg_ref, w_ref, x_ref, st_ref):
    i = pl.program_id(0)
    p = pts_ref[...]
    m = mg_ref[...]
    xy = p[:, 1:3]
    cf = (xy - _PCMIN) / _VOXEL
    coords = jnp.floor(cf)
    center = coords * _VOXEL + _VOXEL / 2.0 + _PCMIN
    feats = jnp.concatenate([p[:, 1:5], p[:, 1:4] - m, xy - center], axis=-1)
    x = jnp.dot(feats, w_ref[...].T, preferred_element_type=jnp.float32)
    x_ref[...] = x
    row = i * _PBLK + jax.lax.broadcasted_iota(jnp.int32, (_PBLK, 1), 0)
    xm = jnp.where(row < _NREAL, x, 0.0)
    st_ref[0, 0, :] = jnp.sum(xm, axis=0)
    st_ref[0, 1, :] = jnp.sum(xm * xm, axis=0)


def _finalize_body(cin_ref, cnt_ref, s_ref, b_ref, out_ref):
    v = cin_ref[0]
    mask = cnt_ref[0] > 0.0
    s = s_ref[0]
    bb = b_ref[0]
    r = jnp.maximum(v * s + bb, 0.0)
    r = jnp.where(mask, r, 0.0)
    vr = r.reshape(_GBLK * _GX, _D)
    t = vr.T
    out_ref[0] = t.reshape(_D, _GBLK, _GX)


def kernel(points, W, gamma, beta):
    n = points.shape[0]
    xy = points[:, 1:3]
    coords = ((xy - _PCMIN) / _VOXEL).astype(jnp.int32)
    bidx = points[:, 0].astype(jnp.int32)
    pidx = bidx * _NV + coords[:, 1] * _GX + coords[:, 0]

    pad = jnp.zeros((_NPAD - n, 5), dtype=jnp.float32).at[:, 0].set(4.0)
    points_padded = jnp.concatenate([points, pad], axis=0)
    table, _pidx_dump = _seg_sum(points_padded)
    cnt = table[3]
    mean = (table[0:3] / jnp.maximum(cnt, 1.0)[None, :]).T
    mean_g = mean[pidx]

    mean_g_pad = jnp.concatenate(
        [mean_g, jnp.zeros((_NPAD - n, 3), dtype=jnp.float32)], axis=0)
    x, stats = pl.pallas_call(
        _matmul_body,
        grid=(_NPAD // _PBLK,),
        in_specs=[
            pl.BlockSpec((_PBLK, 5), lambda i: (i, 0)),
            pl.BlockSpec((_PBLK, 3), lambda i: (i, 0)),
            pl.BlockSpec((_D, 9), lambda i: (0, 0)),
        ],
        out_specs=[
            pl.BlockSpec((_PBLK, _D), lambda i: (i, 0)),
            pl.BlockSpec((1, 2, _D), lambda i: (i, 0, 0)),
        ],
        out_shape=[
            jax.ShapeDtypeStruct((_NPAD, _D), jnp.float32),
            jax.ShapeDtypeStruct((_NPAD // _PBLK, 2, _D), jnp.float32),
        ],
    )(points_padded, mean_g_pad, W)

    tot = jnp.sum(stats, axis=0)
    mu = tot[0] / n
    var = jnp.maximum(tot[1] / n - mu * mu, 0.0)
    s = gamma / jnp.sqrt(var + 1e-3)
    bb = beta - mu * s

    seg_max = jax.ops.segment_max(x[:n], pidx, num_segments=_B * _NV)
    cgrid = seg_max.reshape(_B, _GY, _GX, _D)
    cntg = cnt.reshape(_B, _GY, _GX, 1)

    canvas = pl.pallas_call(
        _finalize_body,
        grid=(_B, _GY // _GBLK),
        in_specs=[
            pl.BlockSpec((1, _GBLK, _GX, _D), lambda b, g: (b, g, 0, 0)),
            pl.BlockSpec((1, _GBLK, _GX, 1), lambda b, g: (b, g, 0, 0)),
            pl.BlockSpec((1, _D), lambda b, g: (0, 0)),
            pl.BlockSpec((1, _D), lambda b, g: (0, 0)),
        ],
        out_specs=pl.BlockSpec((1, _D, _GBLK, _GX), lambda b, g: (b, 0, g, 0)),
        out_shape=jax.ShapeDtypeStruct((_B, _D, _GY, _GX), jnp.float32),
    )(cgrid, cntg, s.reshape(1, _D), bb.reshape(1, _D))

    return canvas


# speedup vs baseline: 1.7192x; 1.0202x over previous
import functools

import jax
import jax.numpy as jnp
from jax import lax
from jax.experimental import pallas as pl
from jax.experimental.pallas import tpu as pltpu
from jax.experimental.pallas import tpu_sc as plsc

_B = 2
_GX = 512
_GY = 512
_NV = _GX * _GY
_D = 64
_VOXEL = 0.2
_PCMIN = -51.2

_PBLK = 4096
_GBLK = 8

_NREAL = 400000
_NPAD = 401408
_HALF = _B * _NV // 2
_TRASH = _HALF
_HROWS = _HALF + 256
_CH = 1568
_SUB = 112
_NCHUNK = _NPAD // 16 // _CH
_ZROWS = _HROWS // 16 // 4


def _seg_sum_body(pb_hbm, px_hbm, py_hbm, pz_hbm, tbl_hbm, pidx_hbm,
                  bbuf, vx, vy, vz, vo, zbuf, idxbuf, pidxbuf,
                  tx, ty, tz, tc):
    c = lax.axis_index("c")
    s = lax.axis_index("s")
    iota = lax.iota(jnp.int32, 16)
    ones16 = jnp.full((16,), 1.0, dtype=jnp.float32)
    zeros16 = jnp.zeros((16,), dtype=jnp.float32)

    def ones_body(g, _):
        vo[pl.ds(g * 16, 16)] = ones16
        return 0

    lax.fori_loop(0, _CH // 16, ones_body, 0)

    def zeros_body(g, _):
        zbuf[pl.ds(g * 16, 16)] = zeros16
        return 0

    lax.fori_loop(0, _HROWS // 16 // 16, zeros_body, 0)

    zoff = s * (_HROWS // 16)
    for t in (tx, ty, tz, tc):
        pltpu.sync_copy(zbuf, t.at[pl.ds(zoff, _HROWS // 16)])
    plsc.subcore_barrier()

    base = s * (_NPAD // 16)
    qbase = c * _HALF

    def chunk_body(ci, _):
        start = base + ci * _CH
        sl_in = pl.ds(start, _CH)
        pltpu.sync_copy(pb_hbm.at[sl_in], bbuf)
        pltpu.sync_copy(px_hbm.at[sl_in], vx)
        pltpu.sync_copy(py_hbm.at[sl_in], vy)
        pltpu.sync_copy(pz_hbm.at[sl_in], vz)

        def group_body(g, _):
            sl16 = pl.ds(g * 16, 16)
            b = bbuf[sl16]
            x = vx[sl16]
            y = vy[sl16]
            cx = ((x - _PCMIN) / _VOXEL).astype(jnp.int32)
            cy = ((y - _PCMIN) / _VOXEL).astype(jnp.int32)
            bi = b.astype(jnp.int32)
            pidx = bi * _NV + cy * _GX + cx
            local = pidx - qbase
            inr = (local >= 0) & (local < _HALF)
            localc = jnp.where(inr, local, _TRASH)
            pidxbuf[sl16] = pidx
            idxbuf[g // 7, pl.ds((g % 7) * 16, 16)] = localc
            return 0

        lax.fori_loop(0, _CH // 16, group_body, 0)

        for j in range(_CH // _SUB):
            sl = pl.ds(j * _SUB, _SUB)
            idxrow = idxbuf.at[j]
            pltpu.sync_copy(vx.at[sl], tx.at[idxrow], add=True)
            pltpu.sync_copy(vy.at[sl], ty.at[idxrow], add=True)
            pltpu.sync_copy(vz.at[sl], tz.at[idxrow], add=True)
            pltpu.sync_copy(vo.at[sl], tc.at[idxrow], add=True)

        @pl.when(c == 0)
        def _():
            pltpu.sync_copy(pidxbuf, pidx_hbm.at[pl.ds(start, _CH)])
        return 0

    lax.fori_loop(0, _NCHUNK, chunk_body, 0)
    plsc.subcore_barrier()

    wrows = _HALF // 16
    for k, t in enumerate((tx, ty, tz, tc)):
        pltpu.sync_copy(t.at[pl.ds(s * wrows, wrows)],
                        tbl_hbm.at[k, pl.ds(qbase + s * wrows, wrows)])


def _seg_sum(points_padded):
    mesh = plsc.VectorSubcoreMesh(core_axis_name="c", subcore_axis_name="s")
    f = functools.partial(
        pl.kernel,
        mesh=mesh,
        out_type=[
            jax.ShapeDtypeStruct((4, _B * _NV), jnp.float32),
            jax.ShapeDtypeStruct((_NPAD,), jnp.int32),
        ],
        scratch_types=[
            pltpu.VMEM((_CH,), jnp.float32),
            pltpu.VMEM((_CH,), jnp.float32),
            pltpu.VMEM((_CH,), jnp.float32),
            pltpu.VMEM((_CH,), jnp.float32),
            pltpu.VMEM((_CH,), jnp.float32),
            pltpu.VMEM((_HROWS // 16,), jnp.float32),
            pltpu.VMEM((_CH // _SUB, _SUB), jnp.int32),
            pltpu.VMEM((_CH,), jnp.int32),
            pltpu.VMEM_SHARED((_HROWS,), jnp.float32),
            pltpu.VMEM_SHARED((_HROWS,), jnp.float32),
            pltpu.VMEM_SHARED((_HROWS,), jnp.float32),
            pltpu.VMEM_SHARED((_HROWS,), jnp.float32),
        ],
    )(_seg_sum_body)
    return f(points_padded[:, 0], points_padded[:, 1],
             points_padded[:, 2], points_padded[:, 3])


def _matmul_body(pts_ref, mg_ref, w_ref, x_ref, st_ref):
    i = pl.program_id(0)
    p = pts_ref[...]
    m = mg_ref[...]
    xy = p[:, 1:3]
    cf = (xy - _PCMIN) / _VOXEL
    coords = jnp.floor(cf)
    center = coords * _VOXEL + _VOXEL / 2.0 + _PCMIN
    feats = jnp.concatenate([p[:, 1:5], p[:, 1:4] - m, xy - center], axis=-1)
    x = jnp.dot(feats, w_ref[...].T, preferred_element_type=jnp.float32)
    x_ref[...] = x
    row = i * _PBLK + jax.lax.broadcasted_iota(jnp.int32, (_PBLK, 1), 0)
    xm = jnp.where(row < _NREAL, x, 0.0)
    st_ref[0, 0, :] = jnp.sum(xm, axis=0)
    st_ref[0, 1, :] = jnp.sum(xm * xm, axis=0)


def _finalize_body(cin_ref, cnt_ref, s_ref, b_ref, out_ref):
    v = cin_ref[0]
    mask = cnt_ref[0] > 0.0
    s = s_ref[0]
    bb = b_ref[0]
    r = jnp.maximum(v * s + bb, 0.0)
    r = jnp.where(mask, r, 0.0)
    vr = r.reshape(_GBLK * _GX, _D)
    t = vr.T
    out_ref[0] = t.reshape(_D, _GBLK, _GX)


def kernel(points, W, gamma, beta):
    n = points.shape[0]
    xy = points[:, 1:3]
    coords = ((xy - _PCMIN) / _VOXEL).astype(jnp.int32)
    bidx = points[:, 0].astype(jnp.int32)
    pidx = bidx * _NV + coords[:, 1] * _GX + coords[:, 0]

    pad = jnp.zeros((_NPAD - n, 5), dtype=jnp.float32).at[:, 0].set(4.0)
    points_padded = jnp.concatenate([points, pad], axis=0)
    table, _pidx_dump = _seg_sum(points_padded)
    cnt = table[3]
    mean = (table[0:3] / jnp.maximum(cnt, 1.0)[None, :]).T
    mean_g = mean[pidx]

    mean_g_pad = jnp.concatenate(
        [mean_g, jnp.zeros((_NPAD - n, 3), dtype=jnp.float32)], axis=0)
    x, stats = pl.pallas_call(
        _matmul_body,
        grid=(_NPAD // _PBLK,),
        in_specs=[
            pl.BlockSpec((_PBLK, 5), lambda i: (i, 0)),
            pl.BlockSpec((_PBLK, 3), lambda i: (i, 0)),
            pl.BlockSpec((_D, 9), lambda i: (0, 0)),
        ],
        out_specs=[
            pl.BlockSpec((_PBLK, _D), lambda i: (i, 0)),
            pl.BlockSpec((1, 2, _D), lambda i: (i, 0, 0)),
        ],
        out_shape=[
            jax.ShapeDtypeStruct((_NPAD, _D), jnp.float32),
            jax.ShapeDtypeStruct((_NPAD // _PBLK, 2, _D), jnp.float32),
        ],
    )(points_padded, mean_g_pad, W)

    tot = jnp.sum(stats, axis=0)
    mu = tot[0] / n
    var = jnp.maximum(tot[1] / n - mu * mu, 0.0)
    s = gamma / jnp.sqrt(var + 1e-3)
    bb = beta - mu * s

    pidx_pad = jnp.concatenate(
        [pidx, jnp.full((_NPAD - n,), _B * _NV + 7, dtype=jnp.int32)])
    seg_max = jax.ops.segment_max(x, pidx_pad, num_segments=_B * _NV)
    cgrid = seg_max.reshape(_B, _GY, _GX, _D)
    cntg = cnt.reshape(_B, _GY, _GX, 1)

    canvas = pl.pallas_call(
        _finalize_body,
        grid=(_B, _GY // _GBLK),
        in_specs=[
            pl.BlockSpec((1, _GBLK, _GX, _D), lambda b, g: (b, g, 0, 0)),
            pl.BlockSpec((1, _GBLK, _GX, 1), lambda b, g: (b, g, 0, 0)),
            pl.BlockSpec((1, _D), lambda b, g: (0, 0)),
            pl.BlockSpec((1, _D), lambda b, g: (0, 0)),
        ],
        out_specs=pl.BlockSpec((1, _D, _GBLK, _GX), lambda b, g: (b, 0, g, 0)),
        out_shape=jax.ShapeDtypeStruct((_B, _D, _GY, _GX), jnp.float32),
    )(cgrid, cntg, s.reshape(1, _D), bb.reshape(1, _D))

    return canvas


# speedup vs baseline: 1.7385x; 1.0112x over previous
import functools

import jax
import jax.numpy as jnp
from jax import lax
from jax.experimental import pallas as pl
from jax.experimental.pallas import tpu as pltpu
from jax.experimental.pallas import tpu_sc as plsc

_B = 2
_GX = 512
_GY = 512
_NV = _GX * _GY
_D = 64
_VOXEL = 0.2
_PCMIN = -51.2

_PBLK = 8192
_GBLK = 16

_NREAL = 400000
_NPAD = 401408
_HALF = _B * _NV // 2
_TRASH = _HALF
_HROWS = _HALF + 256
_CH = 1568
_SUB = 112
_NCHUNK = _NPAD // 16 // _CH
_ZROWS = _HROWS // 16 // 4


def _seg_sum_body(pb_hbm, px_hbm, py_hbm, pz_hbm, tbl_hbm, pidx_hbm,
                  bbuf, vx, vy, vz, vo, zbuf, idxbuf, pidxbuf,
                  tx, ty, tz, tc):
    c = lax.axis_index("c")
    s = lax.axis_index("s")
    iota = lax.iota(jnp.int32, 16)
    ones16 = jnp.full((16,), 1.0, dtype=jnp.float32)
    zeros16 = jnp.zeros((16,), dtype=jnp.float32)

    def ones_body(g, _):
        vo[pl.ds(g * 16, 16)] = ones16
        return 0

    lax.fori_loop(0, _CH // 16, ones_body, 0)

    def zeros_body(g, _):
        zbuf[pl.ds(g * 16, 16)] = zeros16
        return 0

    lax.fori_loop(0, _HROWS // 16 // 16, zeros_body, 0)

    zoff = s * (_HROWS // 16)
    for t in (tx, ty, tz, tc):
        pltpu.sync_copy(zbuf, t.at[pl.ds(zoff, _HROWS // 16)])
    plsc.subcore_barrier()

    base = s * (_NPAD // 16)
    qbase = c * _HALF

    def chunk_body(ci, _):
        start = base + ci * _CH
        sl_in = pl.ds(start, _CH)
        pltpu.sync_copy(pb_hbm.at[sl_in], bbuf)
        pltpu.sync_copy(px_hbm.at[sl_in], vx)
        pltpu.sync_copy(py_hbm.at[sl_in], vy)
        pltpu.sync_copy(pz_hbm.at[sl_in], vz)

        def group_body(g, _):
            sl16 = pl.ds(g * 16, 16)
            b = bbuf[sl16]
            x = vx[sl16]
            y = vy[sl16]
            cx = ((x - _PCMIN) / _VOXEL).astype(jnp.int32)
            cy = ((y - _PCMIN) / _VOXEL).astype(jnp.int32)
            bi = b.astype(jnp.int32)
            pidx = bi * _NV + cy * _GX + cx
            local = pidx - qbase
            inr = (local >= 0) & (local < _HALF)
            localc = jnp.where(inr, local, _TRASH)
            pidxbuf[sl16] = pidx
            idxbuf[g // 7, pl.ds((g % 7) * 16, 16)] = localc
            return 0

        lax.fori_loop(0, _CH // 16, group_body, 0)

        for j in range(_CH // _SUB):
            sl = pl.ds(j * _SUB, _SUB)
            idxrow = idxbuf.at[j]
            pltpu.sync_copy(vx.at[sl], tx.at[idxrow], add=True)
            pltpu.sync_copy(vy.at[sl], ty.at[idxrow], add=True)
            pltpu.sync_copy(vz.at[sl], tz.at[idxrow], add=True)
            pltpu.sync_copy(vo.at[sl], tc.at[idxrow], add=True)

        @pl.when(c == 0)
        def _():
            pltpu.sync_copy(pidxbuf, pidx_hbm.at[pl.ds(start, _CH)])
        return 0

    lax.fori_loop(0, _NCHUNK, chunk_body, 0)
    plsc.subcore_barrier()

    wrows = _HALF // 16
    for k, t in enumerate((tx, ty, tz, tc)):
        pltpu.sync_copy(t.at[pl.ds(s * wrows, wrows)],
                        tbl_hbm.at[k, pl.ds(qbase + s * wrows, wrows)])


def _seg_sum(points_padded):
    mesh = plsc.VectorSubcoreMesh(core_axis_name="c", subcore_axis_name="s")
    f = functools.partial(
        pl.kernel,
        mesh=mesh,
        out_type=[
            jax.ShapeDtypeStruct((4, _B * _NV), jnp.float32),
            jax.ShapeDtypeStruct((_NPAD,), jnp.int32),
        ],
        scratch_types=[
            pltpu.VMEM((_CH,), jnp.float32),
            pltpu.VMEM((_CH,), jnp.float32),
            pltpu.VMEM((_CH,), jnp.float32),
            pltpu.VMEM((_CH,), jnp.float32),
            pltpu.VMEM((_CH,), jnp.float32),
            pltpu.VMEM((_HROWS // 16,), jnp.float32),
            pltpu.VMEM((_CH // _SUB, _SUB), jnp.int32),
            pltpu.VMEM((_CH,), jnp.int32),
            pltpu.VMEM_SHARED((_HROWS,), jnp.float32),
            pltpu.VMEM_SHARED((_HROWS,), jnp.float32),
            pltpu.VMEM_SHARED((_HROWS,), jnp.float32),
            pltpu.VMEM_SHARED((_HROWS,), jnp.float32),
        ],
    )(_seg_sum_body)
    return f(points_padded[:, 0], points_padded[:, 1],
             points_padded[:, 2], points_padded[:, 3])


def _matmul_body(pts_ref, mg_ref, w_ref, x_ref, st_ref):
    i = pl.program_id(0)
    p = pts_ref[...]
    m = mg_ref[...]
    xy = p[:, 1:3]
    cf = (xy - _PCMIN) / _VOXEL
    coords = jnp.floor(cf)
    center = coords * _VOXEL + _VOXEL / 2.0 + _PCMIN
    feats = jnp.concatenate([p[:, 1:5], p[:, 1:4] - m, xy - center], axis=-1)
    x = jnp.dot(feats, w_ref[...].T, preferred_element_type=jnp.float32)
    x_ref[...] = x
    row = i * _PBLK + jax.lax.broadcasted_iota(jnp.int32, (_PBLK, 1), 0)
    xm = jnp.where(row < _NREAL, x, 0.0)
    st_ref[0, 0, :] = jnp.sum(xm, axis=0)
    st_ref[0, 1, :] = jnp.sum(xm * xm, axis=0)


def _finalize_body(cin_ref, cnt_ref, s_ref, b_ref, out_ref):
    v = cin_ref[0]
    mask = cnt_ref[0] > 0.0
    s = s_ref[0]
    bb = b_ref[0]
    r = jnp.maximum(v * s + bb, 0.0)
    r = jnp.where(mask, r, 0.0)
    vr = r.reshape(_GBLK * _GX, _D)
    t = vr.T
    out_ref[0] = t.reshape(_D, _GBLK, _GX)


def kernel(points, W, gamma, beta):
    n = points.shape[0]
    xy = points[:, 1:3]
    coords = ((xy - _PCMIN) / _VOXEL).astype(jnp.int32)
    bidx = points[:, 0].astype(jnp.int32)
    pidx = bidx * _NV + coords[:, 1] * _GX + coords[:, 0]

    pad = jnp.zeros((_NPAD - n, 5), dtype=jnp.float32).at[:, 0].set(4.0)
    points_padded = jnp.concatenate([points, pad], axis=0)
    table, _pidx_dump = _seg_sum(points_padded)
    cnt = table[3]
    mean = (table[0:3] / jnp.maximum(cnt, 1.0)[None, :]).T
    mean_g = mean[pidx]

    mean_g_pad = jnp.concatenate(
        [mean_g, jnp.zeros((_NPAD - n, 3), dtype=jnp.float32)], axis=0)
    x, stats = pl.pallas_call(
        _matmul_body,
        grid=(_NPAD // _PBLK,),
        in_specs=[
            pl.BlockSpec((_PBLK, 5), lambda i: (i, 0)),
            pl.BlockSpec((_PBLK, 3), lambda i: (i, 0)),
            pl.BlockSpec((_D, 9), lambda i: (0, 0)),
        ],
        out_specs=[
            pl.BlockSpec((_PBLK, _D), lambda i: (i, 0)),
            pl.BlockSpec((1, 2, _D), lambda i: (i, 0, 0)),
        ],
        out_shape=[
            jax.ShapeDtypeStruct((_NPAD, _D), jnp.float32),
            jax.ShapeDtypeStruct((_NPAD // _PBLK, 2, _D), jnp.float32),
        ],
    )(points_padded, mean_g_pad, W)

    tot = jnp.sum(stats, axis=0)
    mu = tot[0] / n
    var = jnp.maximum(tot[1] / n - mu * mu, 0.0)
    s = gamma / jnp.sqrt(var + 1e-3)
    bb = beta - mu * s

    pidx_pad = jnp.concatenate(
        [pidx, jnp.full((_NPAD - n,), _B * _NV + 7, dtype=jnp.int32)])
    seg_max = jax.ops.segment_max(x, pidx_pad, num_segments=_B * _NV)
    cgrid = seg_max.reshape(_B, _GY, _GX, _D)
    cntg = cnt.reshape(_B, _GY, _GX, 1)

    canvas = pl.pallas_call(
        _finalize_body,
        grid=(_B, _GY // _GBLK),
        in_specs=[
            pl.BlockSpec((1, _GBLK, _GX, _D), lambda b, g: (b, g, 0, 0)),
            pl.BlockSpec((1, _GBLK, _GX, 1), lambda b, g: (b, g, 0, 0)),
            pl.BlockSpec((1, _D), lambda b, g: (0, 0)),
            pl.BlockSpec((1, _D), lambda b, g: (0, 0)),
        ],
        out_specs=pl.BlockSpec((1, _D, _GBLK, _GX), lambda b, g: (b, 0, g, 0)),
        out_shape=jax.ShapeDtypeStruct((_B, _D, _GY, _GX), jnp.float32),
    )(cgrid, cntg, s.reshape(1, _D), bb.reshape(1, _D))

    return canvas


# speedup vs baseline: 1.7424x; 1.0023x over previous
import functools

import jax
import jax.numpy as jnp
from jax import lax
from jax.experimental import pallas as pl
from jax.experimental.pallas import tpu as pltpu
from jax.experimental.pallas import tpu_sc as plsc

_B = 2
_GX = 512
_GY = 512
_NV = _GX * _GY
_D = 64
_VOXEL = 0.2
_PCMIN = -51.2

_PBLK = 8192
_GBLK = 32

_NREAL = 400000
_NPAD = 401408
_HALF = _B * _NV // 2
_TRASH = _HALF
_HROWS = _HALF + 256
_CH = 1568
_SUB = 112
_NCHUNK = _NPAD // 16 // _CH
_ZROWS = _HROWS // 16 // 4


def _seg_sum_body(pb_hbm, px_hbm, py_hbm, pz_hbm, tbl_hbm, pidx_hbm,
                  bbuf, vx, vy, vz, vo, zbuf, idxbuf, pidxbuf,
                  tx, ty, tz, tc):
    c = lax.axis_index("c")
    s = lax.axis_index("s")
    iota = lax.iota(jnp.int32, 16)
    ones16 = jnp.full((16,), 1.0, dtype=jnp.float32)
    zeros16 = jnp.zeros((16,), dtype=jnp.float32)

    def ones_body(g, _):
        vo[pl.ds(g * 16, 16)] = ones16
        return 0

    lax.fori_loop(0, _CH // 16, ones_body, 0)

    def zeros_body(g, _):
        zbuf[pl.ds(g * 16, 16)] = zeros16
        return 0

    lax.fori_loop(0, _HROWS // 16 // 16, zeros_body, 0)

    zoff = s * (_HROWS // 16)
    for t in (tx, ty, tz, tc):
        pltpu.sync_copy(zbuf, t.at[pl.ds(zoff, _HROWS // 16)])
    plsc.subcore_barrier()

    base = s * (_NPAD // 16)
    qbase = c * _HALF

    def chunk_body(ci, _):
        start = base + ci * _CH
        sl_in = pl.ds(start, _CH)
        pltpu.sync_copy(pb_hbm.at[sl_in], bbuf)
        pltpu.sync_copy(px_hbm.at[sl_in], vx)
        pltpu.sync_copy(py_hbm.at[sl_in], vy)
        pltpu.sync_copy(pz_hbm.at[sl_in], vz)

        def group_body(g, _):
            sl16 = pl.ds(g * 16, 16)
            b = bbuf[sl16]
            x = vx[sl16]
            y = vy[sl16]
            cx = ((x - _PCMIN) / _VOXEL).astype(jnp.int32)
            cy = ((y - _PCMIN) / _VOXEL).astype(jnp.int32)
            bi = b.astype(jnp.int32)
            pidx = bi * _NV + cy * _GX + cx
            local = pidx - qbase
            inr = (local >= 0) & (local < _HALF)
            localc = jnp.where(inr, local, _TRASH)
            pidxbuf[sl16] = pidx
            idxbuf[g // 7, pl.ds((g % 7) * 16, 16)] = localc
            return 0

        lax.fori_loop(0, _CH // 16, group_body, 0)

        for j in range(_CH // _SUB):
            sl = pl.ds(j * _SUB, _SUB)
            idxrow = idxbuf.at[j]
            pltpu.sync_copy(vx.at[sl], tx.at[idxrow], add=True)
            pltpu.sync_copy(vy.at[sl], ty.at[idxrow], add=True)
            pltpu.sync_copy(vz.at[sl], tz.at[idxrow], add=True)
            pltpu.sync_copy(vo.at[sl], tc.at[idxrow], add=True)

        @pl.when(c == 0)
        def _():
            pltpu.sync_copy(pidxbuf, pidx_hbm.at[pl.ds(start, _CH)])
        return 0

    lax.fori_loop(0, _NCHUNK, chunk_body, 0)
    plsc.subcore_barrier()

    wrows = _HALF // 16
    for k, t in enumerate((tx, ty, tz, tc)):
        pltpu.sync_copy(t.at[pl.ds(s * wrows, wrows)],
                        tbl_hbm.at[k, pl.ds(qbase + s * wrows, wrows)])


def _seg_sum(points_padded):
    mesh = plsc.VectorSubcoreMesh(core_axis_name="c", subcore_axis_name="s")
    f = functools.partial(
        pl.kernel,
        mesh=mesh,
        out_type=[
            jax.ShapeDtypeStruct((4, _B * _NV), jnp.float32),
            jax.ShapeDtypeStruct((_NPAD,), jnp.int32),
        ],
        scratch_types=[
            pltpu.VMEM((_CH,), jnp.float32),
            pltpu.VMEM((_CH,), jnp.float32),
            pltpu.VMEM((_CH,), jnp.float32),
            pltpu.VMEM((_CH,), jnp.float32),
            pltpu.VMEM((_CH,), jnp.float32),
            pltpu.VMEM((_HROWS // 16,), jnp.float32),
            pltpu.VMEM((_CH // _SUB, _SUB), jnp.int32),
            pltpu.VMEM((_CH,), jnp.int32),
            pltpu.VMEM_SHARED((_HROWS,), jnp.float32),
            pltpu.VMEM_SHARED((_HROWS,), jnp.float32),
            pltpu.VMEM_SHARED((_HROWS,), jnp.float32),
            pltpu.VMEM_SHARED((_HROWS,), jnp.float32),
        ],
    )(_seg_sum_body)
    return f(points_padded[:, 0], points_padded[:, 1],
             points_padded[:, 2], points_padded[:, 3])


def _matmul_body(pts_ref, mg_ref, w_ref, x_ref, st_ref):
    i = pl.program_id(0)
    p = pts_ref[...]
    m = mg_ref[...]
    xy = p[:, 1:3]
    cf = (xy - _PCMIN) / _VOXEL
    coords = jnp.floor(cf)
    center = coords * _VOXEL + _VOXEL / 2.0 + _PCMIN
    feats = jnp.concatenate([p[:, 1:5], p[:, 1:4] - m, xy - center], axis=-1)
    x = jnp.dot(feats, w_ref[...].T, preferred_element_type=jnp.float32)
    x_ref[...] = x
    row = i * _PBLK + jax.lax.broadcasted_iota(jnp.int32, (_PBLK, 1), 0)
    xm = jnp.where(row < _NREAL, x, 0.0)
    st_ref[0, 0, :] = jnp.sum(xm, axis=0)
    st_ref[0, 1, :] = jnp.sum(xm * xm, axis=0)


def _finalize_body(cin_ref, cnt_ref, s_ref, b_ref, out_ref):
    v = cin_ref[0]
    mask = cnt_ref[0] > 0.0
    s = s_ref[0]
    bb = b_ref[0]
    r = jnp.maximum(v * s + bb, 0.0)
    r = jnp.where(mask, r, 0.0)
    vr = r.reshape(_GBLK * _GX, _D)
    t = vr.T
    out_ref[0] = t.reshape(_D, _GBLK, _GX)


def kernel(points, W, gamma, beta):
    n = points.shape[0]
    xy = points[:, 1:3]
    coords = ((xy - _PCMIN) / _VOXEL).astype(jnp.int32)
    bidx = points[:, 0].astype(jnp.int32)
    pidx = bidx * _NV + coords[:, 1] * _GX + coords[:, 0]

    pad = jnp.zeros((_NPAD - n, 5), dtype=jnp.float32).at[:, 0].set(4.0)
    points_padded = jnp.concatenate([points, pad], axis=0)
    table, _pidx_dump = _seg_sum(points_padded)
    cnt = table[3]
    mean = (table[0:3] / jnp.maximum(cnt, 1.0)[None, :]).T
    mean_g = mean[pidx]

    mean_g_pad = jnp.concatenate(
        [mean_g, jnp.zeros((_NPAD - n, 3), dtype=jnp.float32)], axis=0)
    x, stats = pl.pallas_call(
        _matmul_body,
        grid=(_NPAD // _PBLK,),
        in_specs=[
            pl.BlockSpec((_PBLK, 5), lambda i: (i, 0)),
            pl.BlockSpec((_PBLK, 3), lambda i: (i, 0)),
            pl.BlockSpec((_D, 9), lambda i: (0, 0)),
        ],
        out_specs=[
            pl.BlockSpec((_PBLK, _D), lambda i: (i, 0)),
            pl.BlockSpec((1, 2, _D), lambda i: (i, 0, 0)),
        ],
        out_shape=[
            jax.ShapeDtypeStruct((_NPAD, _D), jnp.float32),
            jax.ShapeDtypeStruct((_NPAD // _PBLK, 2, _D), jnp.float32),
        ],
    )(points_padded, mean_g_pad, W)

    tot = jnp.sum(stats, axis=0)
    mu = tot[0] / n
    var = jnp.maximum(tot[1] / n - mu * mu, 0.0)
    s = gamma / jnp.sqrt(var + 1e-3)
    bb = beta - mu * s

    pidx_pad = jnp.concatenate(
        [pidx, jnp.full((_NPAD - n,), _B * _NV + 7, dtype=jnp.int32)])
    seg_max = jax.ops.segment_max(x, pidx_pad, num_segments=_B * _NV)
    cgrid = seg_max.reshape(_B, _GY, _GX, _D)
    cntg = cnt.reshape(_B, _GY, _GX, 1)

    canvas = pl.pallas_call(
        _finalize_body,
        grid=(_B, _GY // _GBLK),
        in_specs=[
            pl.BlockSpec((1, _GBLK, _GX, _D), lambda b, g: (b, g, 0, 0)),
            pl.BlockSpec((1, _GBLK, _GX, 1), lambda b, g: (b, g, 0, 0)),
            pl.BlockSpec((1, _D), lambda b, g: (0, 0)),
            pl.BlockSpec((1, _D), lambda b, g: (0, 0)),
        ],
        out_specs=pl.BlockSpec((1, _D, _GBLK, _GX), lambda b, g: (b, 0, g, 0)),
        out_shape=jax.ShapeDtypeStruct((_B, _D, _GY, _GX), jnp.float32),
    )(cgrid, cntg, s.reshape(1, _D), bb.reshape(1, _D))

    return canvas
